# baseline (speedup 1.0000x reference)
import functools
import math

import jax
import jax.numpy as jnp
import numpy as np
from jax import lax
from jax.experimental import pallas as pl
from jax.experimental.pallas import tpu as pltpu

_LOG_STD_MAX = 2.0
_LOG_STD_MIN = -20.0
_LN2 = math.log(2.0)
_LOG_2PI = math.log(2.0 * math.pi)

_E = 4
_EH = 128
_BH = 256
_ACT = 4
_OBS = 39
_NT = 50
_L = 512
_ACT_LIMIT = 2.0
_MU_COEF = 0.01

_TB = 2048

_SEL = np.zeros((_E * _EH, 8), np.float32)
for _e in range(_E):
    _SEL[_EH * _e:_EH * (_e + 1), _e] = 1.0


def _round_up(x, m):
    return ((x + m - 1) // m) * m


def _softplus(x):
    return jnp.maximum(x, 0.0) + jnp.log1p(jnp.exp(-jnp.abs(x)))


def _actor_kernel(nz_ref, x_ref, ws_ref, sel_ref, bc_ref, out_ref,
                  *, reg_const):
    f32 = jnp.float32
    bf16 = jnp.bfloat16
    relu = lambda z: jnp.maximum(z, 0.0)
    W = ws_ref
    L = _L

    def tmm_l(wr, a):
        return lax.dot_general(wr, a.astype(bf16), (((0,), (1,)), ((), ())),
                               preferred_element_type=f32)

    def tmm_s(wr, a):
        return lax.dot_general(wr, a.astype(bf16), (((0,), (0,)), ((), ())),
                               preferred_element_type=f32)

    x = jnp.pad(x_ref[...].astype(bf16),
                ((0, 0), (_ACT, 128 - _ACT - _OBS - _NT)))
    hT = relu(tmm_l(W[0:128, 0:_BH], x) + bc_ref[0:_BH, 0:1])
    qtT = tmm_l(W[0:128, L:L + _EH], x)
    h2T = relu(tmm_s(W[L:L + _BH, 0:_BH], hT) + bc_ref[0:_BH, 1:2])
    ehidT = relu(tmm_s(W[2 * L:2 * L + _BH, 0:_E * _EH], h2T)
                 + bc_ref[:, 2:3]).astype(bf16)

    kqs = []
    vals = []
    for e in range(_E):
        r4 = 3 * L + _EH * e
        rkv = 4 * L + _EH * e
        c = _EH * e
        eoT = (tmm_s(W[r4:r4 + _EH, c:c + _EH], ehidT[c:c + _EH, :])
               + bc_ref[c:c + _EH, 3:4])
        eob = eoT.astype(bf16)
        kT = tmm_s(W[rkv:rkv + _EH, c:c + _EH], eob)
        vT = tmm_s(W[rkv:rkv + _EH, L + c:L + c + _EH], eob)
        kqs.append((kT * qtT).astype(bf16))
        vals.append(vT)
    s = tmm_s(sel_ref[...], jnp.concatenate(kqs, axis=0))[0:_E, :]
    m = jnp.max(s, axis=0, keepdims=True)
    ex = jnp.exp(s - m)
    wgt = ex / jnp.sum(ex, axis=0, keepdims=True)
    tower = ((wgt[0:1, :] * vals[0] + wgt[1:2, :] * vals[1])
             + (wgt[2:3, :] * vals[2] + wgt[3:4, :] * vals[3]))

    hmlT = relu(tmm_s(W[6 * L:6 * L + _EH, 0:2 * _BH], tower)
                + bc_ref[:, 4:5])
    hmb = hmlT.astype(bf16)
    mu = (tmm_s(W[7 * L:7 * L + _BH, 0:_ACT], hmb[0:_BH, :])
          + bc_ref[0:_ACT, 5:6])
    ls = (tmm_s(W[7 * L + _BH:8 * L, L:L + _ACT], hmb[_BH:2 * _BH, :])
          + bc_ref[_ACT:2 * _ACT, 5:6])
    ls = jnp.clip(ls, _LOG_STD_MIN, _LOG_STD_MAX)
    nz = nz_ref[0:_ACT, :]

    a = mu + jnp.exp(ls) * nz
    pi = _ACT_LIMIT * jnp.tanh(a)

    logp_elem = -0.5 * nz * nz - ls - 0.5 * _LOG_2PI
    corr = 2.0 * (_LN2 - a - _softplus(-2.0 * a))
    logp = jnp.sum(logp_elem - corr, axis=0, keepdims=True)

    TB = pi.shape[1]
    out_ref[...] = jnp.concatenate(
        [pi, logp, jnp.full((1, TB), reg_const, f32),
         jnp.zeros((2, TB), f32)], axis=0)


def kernel(obs, noise, w):
    B = obs.shape[0]
    obs = obs.astype(jnp.float32)
    noise = noise.astype(jnp.float32)

    TB = _TB
    Bp = _round_up(B, TB)
    if Bp != B:
        obs = jnp.pad(obs, ((0, Bp - B), (0, 0)))
        noise = jnp.pad(noise, ((0, Bp - B), (0, 0)))
    noise_t = jnp.pad(noise.T, ((0, 8 - _ACT), (0, 0)))

    ws = w.astype(jnp.bfloat16)
    bc = (jnp.pad(w[8 * _L:8 * _L + 5, 0:512].T, ((0, 0), (0, 3)))
          .at[0:_ACT, 5].set(w[8 * _L + 5, 0:_ACT])
          .at[_ACT:2 * _ACT, 5].set(w[8 * _L + 5, _L:_L + _ACT]))

    reg_const = -(1.0 / _E) * float(_MU_COEF) * (1.0 + _E * 1e-6)
    kern = functools.partial(_actor_kernel, reg_const=reg_const)

    def full(shape):
        return pl.BlockSpec(shape, lambda *_: (0,) * len(shape))

    out = pl.pallas_call(
        kern,
        out_shape=jax.ShapeDtypeStruct((8, Bp), jnp.float32),
        grid=(Bp // TB,),
        in_specs=[pl.BlockSpec((8, TB), lambda i: (0, i)),
                  pl.BlockSpec((TB, _OBS + _NT), lambda i: (i, 0)),
                  full((8 * _L + 8, 2 * _L)), full((_E * _EH, 8)),
                  full((_L, 8))],
        out_specs=pl.BlockSpec((8, TB), lambda i: (0, i)),
        compiler_params=pltpu.CompilerParams(
            dimension_semantics=("parallel",),
            vmem_limit_bytes=32 * 1024 * 1024),
    )(noise_t, obs, ws, jnp.asarray(_SEL, jnp.bfloat16), bc)

    pi = out[:_ACT, :B].T
    logp = out[_ACT, :B]
    reg = out[_ACT + 1, :B]
    return pi, logp, reg


# speedup vs baseline: 1.0250x; 1.0250x over previous
import functools
import math

import jax
import jax.numpy as jnp
import numpy as np
from jax import lax
from jax.experimental import pallas as pl
from jax.experimental.pallas import tpu as pltpu

_LOG_STD_MAX = 2.0
_LOG_STD_MIN = -20.0
_LN2 = math.log(2.0)
_LOG_2PI = math.log(2.0 * math.pi)

_E = 4
_EH = 128
_BH = 256
_ACT = 4
_OBS = 39
_NT = 50
_L = 512
_ACT_LIMIT = 2.0
_MU_COEF = 0.01

_TB = 2048

_SEL = np.zeros((_E * _EH, 8), np.float32)
for _e in range(_E):
    _SEL[_EH * _e:_EH * (_e + 1), _e] = 1.0


def _round_up(x, m):
    return ((x + m - 1) // m) * m


def _softplus(x):
    return jnp.maximum(x, 0.0) + jnp.log1p(jnp.exp(-jnp.abs(x)))


def _actor_kernel(nz_ref, x_ref, ws_ref, sel_ref, bh_ref, bc_ref, out_ref,
                  *, reg_const):
    f32 = jnp.float32
    bf16 = jnp.bfloat16
    relu = lambda z: jnp.maximum(z, 0.0)
    W = ws_ref
    L = _L

    def tmm_l(wr, a):
        return lax.dot_general(wr, a.astype(bf16), (((0,), (1,)), ((), ())),
                               preferred_element_type=f32)

    def tmm_s(wr, a):
        return lax.dot_general(wr, a.astype(bf16), (((0,), (0,)), ((), ())),
                               preferred_element_type=f32)

    x = jnp.pad(x_ref[...].astype(bf16),
                ((0, 0), (_ACT, 128 - _ACT - _OBS - _NT)))
    hT = relu(tmm_l(W[0:128, 0:_BH], x) + bc_ref[0:_BH, 0:1])
    qtT = tmm_l(W[0:128, L:L + _EH], x)
    h2T = relu(tmm_s(W[L:L + _BH, 0:_BH], hT) + bc_ref[0:_BH, 1:2])
    ehidT = relu(tmm_s(W[2 * L:2 * L + _BH, 0:_E * _EH], h2T)
                 + bc_ref[:, 2:3]).astype(bf16)

    kqs = []
    vals = []
    for e in range(_E):
        r4 = 3 * L + _EH * e
        rkv = 4 * L + _EH * e
        c = _EH * e
        eoT = (tmm_s(W[r4:r4 + _EH, c:c + _EH], ehidT[c:c + _EH, :])
               + bc_ref[c:c + _EH, 3:4])
        eob = eoT.astype(bf16)
        kT = tmm_s(W[rkv:rkv + _EH, c:c + _EH], eob)
        vT = tmm_s(W[rkv:rkv + _EH, L + c:L + c + _EH], eob)
        kqs.append((kT * qtT).astype(bf16))
        vals.append(vT)
    s = tmm_s(sel_ref[...], jnp.concatenate(kqs, axis=0))[0:_E, :]
    m = jnp.max(s, axis=0, keepdims=True)
    ex = jnp.exp(s - m)
    wgt = ex / jnp.sum(ex, axis=0, keepdims=True)
    tower = ((wgt[0:1, :] * vals[0] + wgt[1:2, :] * vals[1])
             + (wgt[2:3, :] * vals[2] + wgt[3:4, :] * vals[3]))

    hmlT = relu(tmm_s(W[6 * L:6 * L + _EH, 0:2 * _BH], tower)
                + bc_ref[:, 4:5])
    hmb = hmlT.astype(bf16)
    mu = (tmm_s(W[7 * L:7 * L + _BH, 0:_ACT], hmb[0:_BH, :])
          + bh_ref[0:_ACT, 0:1])
    ls = (tmm_s(W[7 * L + _BH:8 * L, L:L + _ACT], hmb[_BH:2 * _BH, :])
          + bh_ref[8:8 + _ACT, 0:1])
    ls = jnp.clip(ls, _LOG_STD_MIN, _LOG_STD_MAX)
    nz = nz_ref[0:_ACT, :]

    a = mu + jnp.exp(ls) * nz
    pi = _ACT_LIMIT * jnp.tanh(a)

    logp_elem = -0.5 * nz * nz - ls - 0.5 * _LOG_2PI
    corr = 2.0 * (_LN2 - a - _softplus(-2.0 * a))
    logp = jnp.sum(logp_elem - corr, axis=0, keepdims=True)

    TB = pi.shape[1]
    out_ref[...] = jnp.concatenate(
        [pi, logp, jnp.full((1, TB), reg_const, f32),
         jnp.zeros((2, TB), f32)], axis=0)


def kernel(obs, noise, w):
    B = obs.shape[0]
    obs = obs.astype(jnp.float32)
    noise = noise.astype(jnp.float32)

    TB = _TB
    Bp = _round_up(B, TB)
    if Bp != B:
        obs = jnp.pad(obs, ((0, Bp - B), (0, 0)))
        noise = jnp.pad(noise, ((0, Bp - B), (0, 0)))
    noise_t = jnp.pad(noise.T, ((0, 8 - _ACT), (0, 0)))

    ws = w.astype(jnp.bfloat16)
    bc = jnp.pad(w[8 * _L:8 * _L + 5, 0:512].T, ((0, 0), (0, 3)))
    bh = (jnp.zeros((16, 128), jnp.float32)
          .at[0:_ACT, 0].set(w[8 * _L + 5, 0:_ACT])
          .at[8:8 + _ACT, 0].set(w[8 * _L + 5, _L:_L + _ACT]))

    reg_const = -(1.0 / _E) * float(_MU_COEF) * (1.0 + _E * 1e-6)
    kern = functools.partial(_actor_kernel, reg_const=reg_const)

    def full(shape):
        return pl.BlockSpec(shape, lambda *_: (0,) * len(shape))

    out = pl.pallas_call(
        kern,
        out_shape=jax.ShapeDtypeStruct((8, Bp), jnp.float32),
        grid=(Bp // TB,),
        in_specs=[pl.BlockSpec((8, TB), lambda i: (0, i)),
                  pl.BlockSpec((TB, _OBS + _NT), lambda i: (i, 0)),
                  full((8 * _L + 8, 2 * _L)), full((_E * _EH, 8)),
                  full((16, 128)), full((_L, 8))],
        out_specs=pl.BlockSpec((8, TB), lambda i: (0, i)),
        compiler_params=pltpu.CompilerParams(
            dimension_semantics=("parallel",),
            vmem_limit_bytes=32 * 1024 * 1024),
    )(noise_t, obs, ws, jnp.asarray(_SEL, jnp.bfloat16), bh, bc)

    pi = out[:_ACT, :B].T
    logp = out[_ACT, :B]
    reg = out[_ACT + 1, :B]
    return pi, logp, reg


# speedup vs baseline: 1.0259x; 1.0009x over previous
import functools
import math

import jax
import jax.numpy as jnp
import numpy as np
from jax import lax
from jax.experimental import pallas as pl
from jax.experimental.pallas import tpu as pltpu

_LOG_STD_MAX = 2.0
_LOG_STD_MIN = -20.0
_LN2 = math.log(2.0)
_LOG_2PI = math.log(2.0 * math.pi)

_E = 4
_EH = 128
_BH = 256
_ACT = 4
_OBS = 39
_NT = 50
_L = 512
_ACT_LIMIT = 2.0
_MU_COEF = 0.01

_TB = 2048

_SEL = np.zeros((_E * _EH, 8), np.float32)
for _e in range(_E):
    _SEL[_EH * _e:_EH * (_e + 1), _e] = 1.0


def _round_up(x, m):
    return ((x + m - 1) // m) * m


def _softplus(x):
    return jnp.maximum(x, 0.0) + jnp.log1p(jnp.exp(-jnp.abs(x)))


def _actor_kernel(nz_ref, x_ref, ws_ref, sel_ref, bh_ref, bc_ref, out_ref,
                  *, reg_const):
    f32 = jnp.float32
    bf16 = jnp.bfloat16
    relu = lambda z: jnp.maximum(z, 0.0)
    W = ws_ref
    L = _L

    def tmm_l(wr, a):
        return lax.dot_general(wr, a.astype(bf16), (((0,), (1,)), ((), ())),
                               preferred_element_type=f32)

    def tmm_s(wr, a):
        return lax.dot_general(wr, a.astype(bf16), (((0,), (0,)), ((), ())),
                               preferred_element_type=f32)

    x = jnp.pad(x_ref[...].astype(bf16),
                ((0, 0), (_ACT, 128 - _ACT - _OBS - _NT)))
    hT = relu(tmm_l(W[0:128, 0:_BH], x) + bc_ref[0:_BH, 0:1])
    qtT = tmm_l(W[0:128, L:L + _EH], x)
    h2T = relu(tmm_s(W[L:L + _BH, 0:_BH], hT) + bc_ref[0:_BH, 1:2])
    ehidT = relu(tmm_s(W[2 * L:2 * L + _BH, 0:_E * _EH], h2T)
                 + bc_ref[:, 2:3]).astype(bf16)

    kqs = []
    vals = []
    for e in range(_E):
        r4 = 3 * L + _EH * e
        rkv = 4 * L + _EH * e
        c = _EH * e
        eoT = (tmm_s(W[r4:r4 + _EH, c:c + _EH], ehidT[c:c + _EH, :])
               + bc_ref[c:c + _EH, 3:4])
        eob = eoT.astype(bf16)
        kT = tmm_s(W[rkv:rkv + _EH, c:c + _EH], eob)
        vT = tmm_s(W[rkv:rkv + _EH, L + c:L + c + _EH], eob)
        kqs.append((kT * qtT).astype(bf16))
        vals.append(vT)
    s = tmm_s(sel_ref[...], jnp.concatenate(kqs, axis=0))[0:_E, :]
    m = jnp.max(s, axis=0, keepdims=True)
    ex = jnp.exp(s - m)
    wgt = ex / jnp.sum(ex, axis=0, keepdims=True)
    tower = ((wgt[0:1, :] * vals[0] + wgt[1:2, :] * vals[1])
             + (wgt[2:3, :] * vals[2] + wgt[3:4, :] * vals[3]))

    hmlT = relu(tmm_s(W[6 * L:6 * L + _EH, 0:2 * _BH], tower)
                + bc_ref[:, 4:5])
    hmb = hmlT.astype(bf16)
    mu = (tmm_s(W[7 * L:7 * L + _BH, 0:_ACT], hmb[0:_BH, :])
          + bh_ref[0:_ACT, 0:1])
    ls = (tmm_s(W[7 * L + _BH:8 * L, L:L + _ACT], hmb[_BH:2 * _BH, :])
          + bh_ref[8:8 + _ACT, 0:1])
    ls = jnp.clip(ls, _LOG_STD_MIN, _LOG_STD_MAX)
    nz = nz_ref[0:_ACT, :]

    a = mu + jnp.exp(ls) * nz
    pi = _ACT_LIMIT * jnp.tanh(a)

    logp_elem = -0.5 * nz * nz - ls - 0.5 * _LOG_2PI
    corr = 2.0 * (_LN2 - a - _softplus(-2.0 * a))
    logp = jnp.sum(logp_elem - corr, axis=0, keepdims=True)

    TB = pi.shape[1]
    out_ref[...] = jnp.concatenate(
        [pi, logp, jnp.full((1, TB), reg_const, f32),
         jnp.zeros((2, TB), f32)], axis=0)


def kernel(obs, noise, w):
    B = obs.shape[0]
    obs = obs.astype(jnp.float32)
    noise = noise.astype(jnp.float32)

    TB = _TB
    Bp = _round_up(B, TB)
    if Bp != B:
        obs = jnp.pad(obs, ((0, Bp - B), (0, 0)))
        noise = jnp.pad(noise, ((0, Bp - B), (0, 0)))
    noise_t = jnp.pad(noise.T, ((0, 8 - _ACT), (0, 0)))

    ws = w.astype(jnp.bfloat16)
    bc = jnp.pad(w[8 * _L:8 * _L + 5, 0:512].T, ((0, 0), (0, 3)))
    bh = (jnp.zeros((16, 128), jnp.float32)
          .at[0:_ACT, 0].set(w[8 * _L + 5, 0:_ACT])
          .at[8:8 + _ACT, 0].set(w[8 * _L + 5, _L:_L + _ACT]))

    reg_const = -(1.0 / _E) * float(_MU_COEF) * (1.0 + _E * 1e-6)
    kern = functools.partial(_actor_kernel, reg_const=reg_const)

    def full(shape):
        return pl.BlockSpec(shape, lambda *_: (0,) * len(shape))

    out = pl.pallas_call(
        kern,
        out_shape=jax.ShapeDtypeStruct((8, Bp), jnp.float32),
        grid=(Bp // TB,),
        in_specs=[pl.BlockSpec((8, TB), lambda i: (0, i)),
                  pl.BlockSpec((TB, _OBS + _NT), lambda i: (i, 0)),
                  full((8 * _L + 8, 2 * _L)), full((_E * _EH, 8)),
                  full((16, 128)), full((_L, 8))],
        out_specs=pl.BlockSpec((8, TB), lambda i: (0, i)),
        compiler_params=pltpu.CompilerParams(
            dimension_semantics=("arbitrary",),
            vmem_limit_bytes=32 * 1024 * 1024),
    )(noise_t, obs, ws, jnp.asarray(_SEL, jnp.bfloat16), bh, bc)

    pi = out[:_ACT, :B].T
    logp = out[_ACT, :B]
    reg = out[_ACT + 1, :B]
    return pi, logp, reg
